# baseline (speedup 1.0000x reference)
import functools

import jax
import jax.numpy as jnp
from jax import lax
from jax.experimental import pallas as pl
from jax.experimental.pallas import tpu as pltpu
from jax.experimental.pallas import tpu_sc as plsc

NC, NS, L = 2, 16, 16
NW = NC * NS
BATCH = 16384
DIM = 64
BPW = BATCH // NW
CH = 128
NCH = BPW // CH
GRP = CH // L


def _tec_body(heads, rels, tails, ent, rel, out,
              hidx, ridx, tidx, hbuf, rbuf, tbuf, outv, sem_i, sem_g):
    wid = lax.axis_index("s") * NC + lax.axis_index("c")
    base = wid * BPW

    idx_cps = []
    for c in range(NCH):
        off = base + c * CH
        idx_cps.append(pltpu.async_copy(heads.at[pl.ds(off, CH)], hidx.at[c], sem_i))
        idx_cps.append(pltpu.async_copy(rels.at[pl.ds(off, CH)], ridx.at[c], sem_i))
        idx_cps.append(pltpu.async_copy(tails.at[pl.ds(off, CH)], tidx.at[c], sem_i))
    for cp in idx_cps:
        cp.wait()

    gat_cps = []
    for c in range(NCH):
        gat_cps.append(pltpu.async_copy(ent.at[hidx.at[c]], hbuf.at[c], sem_g))
        gat_cps.append(pltpu.async_copy(rel.at[ridx.at[c]], rbuf.at[c], sem_g))
        gat_cps.append(pltpu.async_copy(ent.at[tidx.at[c]], tbuf.at[c], sem_g))
    for cp in gat_cps:
        cp.wait()

    lane = lax.iota(jnp.int32, L)
    perms = [lane ^ sh for sh in (1, 2, 4, 8)]

    for c in range(NCH):
        def grp_body(gi, carry, c=c):
            sq = jnp.zeros((L,), jnp.float32)
            for k in range(L):
                row = gi * L + k
                acc = None
                for q in range(DIM // L):
                    sl = pl.ds(q * L, L)
                    d = hbuf[c, row, sl] + rbuf[c, row, sl] - tbuf[c, row, sl]
                    p = d * d
                    acc = p if acc is None else acc + p
                for pm in perms:
                    acc = acc + acc.at[pm].get(mode="promise_in_bounds")
                sq = jnp.where(lane == k, acc, sq)
            seed = jnp.int32(0x5F3759DF) - (plsc.bitcast(sq, jnp.int32) >> 1)
            y = plsc.bitcast(seed, jnp.float32)
            for _ in range(3):
                y = y * (jnp.float32(1.5) - jnp.float32(0.5) * sq * y * y)
            outv[pl.ds(c * CH + gi * L, L)] = -(sq * y)
            return carry
        lax.fori_loop(0, GRP, grp_body, 0)

    pltpu.sync_copy(outv, out.at[pl.ds(base, BPW)])


_transe = functools.partial(
    pl.kernel,
    out_type=jax.ShapeDtypeStruct((BATCH,), jnp.float32),
    mesh=plsc.VectorSubcoreMesh(core_axis_name="c", subcore_axis_name="s"),
    compiler_params=pltpu.CompilerParams(
        needs_layout_passes=False, use_tc_tiling_on_sc=False),
    scratch_types=[
        pltpu.VMEM((NCH, CH), jnp.int32),
        pltpu.VMEM((NCH, CH), jnp.int32),
        pltpu.VMEM((NCH, CH), jnp.int32),
        pltpu.VMEM((NCH, CH, DIM), jnp.float32),
        pltpu.VMEM((NCH, CH, DIM), jnp.float32),
        pltpu.VMEM((NCH, CH, DIM), jnp.float32),
        pltpu.VMEM((BPW,), jnp.float32),
        pltpu.SemaphoreType.DMA,
        pltpu.SemaphoreType.DMA,
    ],
)(_tec_body)


@jax.jit
def kernel(heads, rels, tails, entity_emb, relation_emb):
    return _transe(heads, rels, tails, entity_emb, relation_emb)
